# baseline (speedup 1.0000x reference)
import jax
import jax.numpy as jnp
from jax import lax
from jax.experimental import pallas as pl
from jax.experimental.pallas import tpu as pltpu
from jax.experimental.pallas import tpu_sc as plsc

N = 10000
E = 320000
C = 128
EMB = 512
B = 4
NET = 7
AVG_DEG = 32.0
GROUP = 32
CPG = C // GROUP
EPS = 1e-5

NW = 32
CH = 128
EPAD = 327680
NCHUNK = 80
ROWS_I = EPAD // CH
ACC = 10240
TPT = ACC // 16
TRASH = 10200


def _gn_silu(data, bid, gamma, beta):
    r = lax.broadcasted_iota(jnp.int32, (C, C), 0) // CPG
    col = lax.broadcasted_iota(jnp.int32, (C, C), 1) // CPG
    G = (r == col).astype(jnp.float32)

    sums = []
    invc = []
    for b in range(B):
        m = bid == b
        sums.append(jnp.sum(jnp.where(m, data, 0.0), axis=0, keepdims=True))
        cnt = jnp.sum(jnp.where(m, 1.0, 0.0))
        invc.append(1.0 / (cnt * CPG + EPS))
    S = jnp.concatenate(sums, axis=0)
    SG = jnp.dot(S, G, preferred_element_type=jnp.float32)

    mf = SG[B - 1:B] * invc[B - 1]
    for b in range(B - 2, -1, -1):
        mf = jnp.where(bid == b, SG[b:b + 1] * invc[b], mf)
    centered = data - mf

    cc = centered * centered
    vs = [jnp.sum(jnp.where(bid == b, cc, 0.0), axis=0, keepdims=True)
          for b in range(B)]
    V = jnp.concatenate(vs, axis=0)
    VG = jnp.dot(V, G, preferred_element_type=jnp.float32)

    isf = lax.rsqrt(VG[B - 1:B] * invc[B - 1] + EPS)
    for b in range(B - 2, -1, -1):
        isf = jnp.where(bid == b, lax.rsqrt(VG[b:b + 1] * invc[b] + EPS), isf)

    out = centered * isf * gamma + beta
    return out * jax.nn.sigmoid(out)


def _prep_body(x_ref, bid_ref, g1_ref, b1_ref, embp_ref, We_ref, be_ref,
               colp_ref, typep_ref, h1_ref, embout_ref, gidx_ref):
    h1 = _gn_silu(x_ref[...], bid_ref[...], g1_ref[...], b1_ref[...])
    h1_ref[...] = h1.astype(jnp.bfloat16)
    e = embp_ref[...]
    e = e * jax.nn.sigmoid(e)
    embout_ref[...] = (jnp.dot(e, We_ref[...], preferred_element_type=jnp.float32)
                       + be_ref[...])
    gidx_ref[...] = colp_ref[...] * NET + typep_ref[...]


def _mid_body(s1a_ref, s1b_ref, embout_ref, bid_ref, g2_ref, b2_ref, h2_ref):
    bid = bid_ref[...]
    hm = ((s1a_ref[...].astype(jnp.float32) + s1b_ref[...].astype(jnp.float32))
          * (1.0 / AVG_DEG))
    eo = embout_ref[...]
    ef = eo[B - 1:B]
    for b in range(B - 2, -1, -1):
        ef = jnp.where(bid == b, eo[b:b + 1], ef)
    hm = hm + ef
    h2_ref[...] = _gn_silu(hm, bid, g2_ref[...], b2_ref[...]).astype(jnp.bfloat16)


def _final_body(x_ref, s2a_ref, s2b_ref, out_ref):
    out_ref[...] = x_ref[...] + (
        s2a_ref[...].astype(jnp.float32) + s2b_ref[...].astype(jnp.float32)
    ) * (1.0 / AVG_DEG)


def _mm_body(h_ref, w_ref, o_ref):
    o_ref[...] = jnp.dot(h_ref[...], w_ref[...],
                         preferred_element_type=jnp.float32)


_MM_RT = 2000


def _matmul_call(h, wcat):
    return pl.pallas_call(
        _mm_body,
        grid=(N // _MM_RT,),
        in_specs=[
            pl.BlockSpec((_MM_RT, C), lambda i: (i, 0)),
            pl.BlockSpec((C, NET * C), lambda i: (0, 0)),
        ],
        out_specs=pl.BlockSpec((_MM_RT, NET * C), lambda i: (i, 0)),
        out_shape=jax.ShapeDtypeStruct((N, NET * C), jnp.float32),
    )(h, wcat)


STG = 16
NSTG = NCHUNK // STG


NC0 = 48
NC1 = NCHUNK * 2 - NC0


def _sc_edge_body(gidx_hbm, grow_hbm, table_hbm, zrow_hbm, out_hbm,
                  sidx, srow, buf0, buf1, accum, semA, semB, semI, semJ):
    c = lax.axis_index("c")
    s = lax.axis_index("s")
    sbase = pl.multiple_of(s * TPT, CH)

    pltpu.sync_copy(zrow_hbm, buf0)
    for k in range(TPT // CH):
        pltpu.sync_copy(buf0, accum.at[pl.ds(sbase + k * CH, CH)])
    plsc.subcore_barrier()

    def _run_edges(nchunk, region):
        nstg = nchunk // STG
        tbase = region + s * nchunk
        pltpu.sync_copy(gidx_hbm.at[pl.ds(tbase, STG)], sidx.at[0])
        pltpu.sync_copy(grow_hbm.at[pl.ds(tbase, STG)], srow.at[0])

        def _stage(st, carry):
            slot = lax.rem(st, 2)
            nslot = 1 - slot
            nbase = tbase + (st + 1) * STG

            @pl.when(st < nstg - 1)
            def _():
                pltpu.async_copy(gidx_hbm.at[pl.ds(nbase, STG)],
                                 sidx.at[nslot], semI)
                pltpu.async_copy(grow_hbm.at[pl.ds(nbase, STG)],
                                 srow.at[nslot], semJ)

            pltpu.async_copy(table_hbm.at[sidx.at[slot, 0]], buf0, semA)

            def _step(t, carry2):
                i = t * 2
                hb = pltpu.async_copy(table_hbm.at[sidx.at[slot, i + 1]],
                                      buf1, semB)
                pltpu.make_async_copy(table_hbm.at[sidx.at[slot, i]],
                                      buf0, semA).wait()
                pltpu.sync_copy(buf0, accum.at[srow.at[slot, i]], add=True)

                @pl.when(t < STG // 2 - 1)
                def _():
                    pltpu.async_copy(table_hbm.at[sidx.at[slot, i + 2]],
                                     buf0, semA)

                hb.wait()
                pltpu.sync_copy(buf1, accum.at[srow.at[slot, i + 1]],
                                add=True)
                return carry2
            lax.fori_loop(0, STG // 2, _step, 0)

            @pl.when(st < nstg - 1)
            def _():
                pltpu.make_async_copy(gidx_hbm.at[pl.ds(nbase, STG)],
                                      sidx.at[nslot], semI).wait()
                pltpu.make_async_copy(grow_hbm.at[pl.ds(nbase, STG)],
                                      srow.at[nslot], semJ).wait()
            return carry
        lax.fori_loop(0, nstg, _stage, 0)

    @pl.when(c == 0)
    def _():
        _run_edges(NC0, 0)

    @pl.when(c == 1)
    def _():
        _run_edges(NC1, 16 * NC0)

    plsc.subcore_barrier()
    pltpu.sync_copy(accum.at[pl.ds(sbase, TPT)],
                    out_hbm.at[c, pl.ds(sbase, TPT)])


_sc_edge_cache = []


def _edge_accumulate(gidx, rowp, table):
    if not _sc_edge_cache:
        mesh = plsc.VectorSubcoreMesh(core_axis_name="c", subcore_axis_name="s")
        _sc_edge_cache.append(pl.kernel(
            _sc_edge_body,
            out_type=jax.ShapeDtypeStruct((2, ACC, C), jnp.float32),
            mesh=mesh,
            scratch_types=[
                pltpu.VMEM((2, STG, CH), jnp.int32),
                pltpu.VMEM((2, STG, CH), jnp.int32),
                pltpu.VMEM((CH, C), jnp.float32),
                pltpu.VMEM((CH, C), jnp.float32),
                pltpu.VMEM_SHARED((ACC, C), jnp.float32),
                pltpu.SemaphoreType.DMA,
                pltpu.SemaphoreType.DMA,
                pltpu.SemaphoreType.DMA,
                pltpu.SemaphoreType.DMA,
            ],
        ))
    zrow = jnp.zeros((CH, C), jnp.float32)
    return _sc_edge_cache[0](gidx, rowp, table, zrow)


def kernel(x, emb, edge_index, edge_type, batch_id,
           gamma1, beta1, W1, We, be, gamma2, beta2, W2):
    f32 = jnp.float32
    row = edge_index[0]
    colv = edge_index[1]
    pad = EPAD - E
    colp = jnp.concatenate([colv, jnp.zeros((pad,), jnp.int32)]).reshape(ROWS_I, CH)
    typep = jnp.concatenate([edge_type, jnp.zeros((pad,), jnp.int32)]).reshape(ROWS_I, CH)
    rowp = jnp.concatenate([row, jnp.full((pad,), TRASH, jnp.int32)]).reshape(ROWS_I, CH)
    bid2 = batch_id.reshape(N, 1)
    embp = jnp.concatenate([emb, jnp.zeros((8 - B, EMB), f32)], axis=0)
    g1r, b1r = gamma1.reshape(1, C), beta1.reshape(1, C)
    g2r, b2r = gamma2.reshape(1, C), beta2.reshape(1, C)
    ber = be.reshape(1, C)
    W1cat = W1.reshape(NET, C, C).transpose(1, 0, 2).reshape(C, NET * C)
    W2cat = W2.reshape(NET, C, C).transpose(1, 0, 2).reshape(C, NET * C)

    W1b = W1cat.astype(jnp.bfloat16)
    W2b = W2cat.astype(jnp.bfloat16)

    h1, embout, gidx = pl.pallas_call(
        _prep_body,
        out_shape=[
            jax.ShapeDtypeStruct((N, C), jnp.bfloat16),
            jax.ShapeDtypeStruct((8, C), f32),
            jax.ShapeDtypeStruct((ROWS_I, CH), jnp.int32),
        ],
    )(x, bid2, g1r, b1r, embp, We, ber, colp, typep)

    T1 = _matmul_call(h1, W1b)
    S1 = _edge_accumulate(gidx, rowp, T1.reshape(N * NET, C))

    h2 = pl.pallas_call(
        _mid_body,
        out_shape=jax.ShapeDtypeStruct((N, C), jnp.bfloat16),
    )(S1[0, :N], S1[1, :N], embout, bid2, g2r, b2r)

    T2 = _matmul_call(h2, W2b)
    S2 = _edge_accumulate(gidx, rowp, T2.reshape(N * NET, C))

    return pl.pallas_call(
        _final_body,
        out_shape=jax.ShapeDtypeStruct((N, C), f32),
    )(x, S2[0, :N], S2[1, :N])


# speedup vs baseline: 1.0323x; 1.0323x over previous
import jax
import jax.numpy as jnp
from jax import lax
from jax.experimental import pallas as pl
from jax.experimental.pallas import tpu as pltpu
from jax.experimental.pallas import tpu_sc as plsc

N = 10000
E = 320000
C = 128
EMB = 512
B = 4
NET = 7
AVG_DEG = 32.0
GROUP = 32
CPG = C // GROUP
EPS = 1e-5

NW = 32
CH = 128
EPAD = 327680
NCHUNK = 80
ROWS_I = EPAD // CH
ACC = 10240
TPT = ACC // 16
TRASH = 10200


def _gn_silu(data, bid, gamma, beta):
    r = lax.broadcasted_iota(jnp.int32, (C, C), 0) // CPG
    col = lax.broadcasted_iota(jnp.int32, (C, C), 1) // CPG
    G = (r == col).astype(jnp.float32)

    sums = []
    invc = []
    for b in range(B):
        m = bid == b
        sums.append(jnp.sum(jnp.where(m, data, 0.0), axis=0, keepdims=True))
        cnt = jnp.sum(jnp.where(m, 1.0, 0.0))
        invc.append(1.0 / (cnt * CPG + EPS))
    S = jnp.concatenate(sums, axis=0)
    SG = jnp.dot(S, G, preferred_element_type=jnp.float32)

    mf = SG[B - 1:B] * invc[B - 1]
    for b in range(B - 2, -1, -1):
        mf = jnp.where(bid == b, SG[b:b + 1] * invc[b], mf)
    centered = data - mf

    cc = centered * centered
    vs = [jnp.sum(jnp.where(bid == b, cc, 0.0), axis=0, keepdims=True)
          for b in range(B)]
    V = jnp.concatenate(vs, axis=0)
    VG = jnp.dot(V, G, preferred_element_type=jnp.float32)

    isf = lax.rsqrt(VG[B - 1:B] * invc[B - 1] + EPS)
    for b in range(B - 2, -1, -1):
        isf = jnp.where(bid == b, lax.rsqrt(VG[b:b + 1] * invc[b] + EPS), isf)

    out = centered * isf * gamma + beta
    return out * jax.nn.sigmoid(out)


def _prep_body(x_ref, bid_ref, g1_ref, b1_ref, embp_ref, We_ref, be_ref,
               colp_ref, typep_ref, h1_ref, embout_ref, gidx_ref):
    h1 = _gn_silu(x_ref[...], bid_ref[...], g1_ref[...], b1_ref[...])
    h1_ref[...] = h1.astype(jnp.bfloat16)
    e = embp_ref[...]
    e = e * jax.nn.sigmoid(e)
    embout_ref[...] = (jnp.dot(e, We_ref[...], preferred_element_type=jnp.float32)
                       + be_ref[...])
    gidx_ref[...] = colp_ref[...] * NET + typep_ref[...]


def _mid_body(s1a_ref, s1b_ref, embout_ref, bid_ref, g2_ref, b2_ref, h2_ref):
    bid = bid_ref[...]
    hm = ((s1a_ref[...].astype(jnp.float32) + s1b_ref[...].astype(jnp.float32))
          * (1.0 / AVG_DEG))
    eo = embout_ref[...]
    ef = eo[B - 1:B]
    for b in range(B - 2, -1, -1):
        ef = jnp.where(bid == b, eo[b:b + 1], ef)
    hm = hm + ef
    h2_ref[...] = _gn_silu(hm, bid, g2_ref[...], b2_ref[...]).astype(jnp.bfloat16)


def _final_body(x_ref, s2a_ref, s2b_ref, out_ref):
    out_ref[...] = x_ref[...] + (
        s2a_ref[...].astype(jnp.float32) + s2b_ref[...].astype(jnp.float32)
    ) * (1.0 / AVG_DEG)


def _mm_body(h_ref, w_ref, o_ref):
    o_ref[...] = jnp.dot(h_ref[...], w_ref[...],
                         preferred_element_type=jnp.float32)


_MM_RT = 2000


def _matmul_call(h, wcat):
    return pl.pallas_call(
        _mm_body,
        grid=(N // _MM_RT,),
        in_specs=[
            pl.BlockSpec((_MM_RT, C), lambda i: (i, 0)),
            pl.BlockSpec((C, NET * C), lambda i: (0, 0)),
        ],
        out_specs=pl.BlockSpec((_MM_RT, NET * C), lambda i: (i, 0)),
        out_shape=jax.ShapeDtypeStruct((N, NET * C), jnp.float32),
    )(h, wcat)


STG = 16
NSTG = NCHUNK // STG


def _sc_edge_body(gidx_hbm, grow_hbm, table_hbm, zrow_hbm, out_hbm,
                  sidx, srow, buf0, buf1, accum, semA, semB, semI, semJ):
    c = lax.axis_index("c")
    s = lax.axis_index("s")
    g = s * 2 + c
    sbase = pl.multiple_of(s * TPT, CH)

    pltpu.sync_copy(zrow_hbm, buf0)
    for k in range(TPT // CH):
        pltpu.sync_copy(buf0, accum.at[pl.ds(sbase + k * CH, CH)])
    plsc.subcore_barrier()

    pltpu.sync_copy(gidx_hbm.at[pl.ds(g * NCHUNK, STG)], sidx.at[0])
    pltpu.sync_copy(grow_hbm.at[pl.ds(g * NCHUNK, STG)], srow.at[0])

    def _stage(st, carry):
        slot = lax.rem(st, 2)
        nslot = 1 - slot
        nbase = g * NCHUNK + (st + 1) * STG

        @pl.when(st < NSTG - 1)
        def _():
            pltpu.async_copy(gidx_hbm.at[pl.ds(nbase, STG)],
                             sidx.at[nslot], semI)
            pltpu.async_copy(grow_hbm.at[pl.ds(nbase, STG)],
                             srow.at[nslot], semJ)

        pltpu.async_copy(table_hbm.at[sidx.at[slot, 0]], buf0, semA)

        def _step(t, carry2):
            i = t * 2
            hb = pltpu.async_copy(table_hbm.at[sidx.at[slot, i + 1]],
                                  buf1, semB)
            pltpu.make_async_copy(table_hbm.at[sidx.at[slot, i]],
                                  buf0, semA).wait()
            pltpu.sync_copy(buf0, accum.at[srow.at[slot, i]], add=True)

            @pl.when(t < STG // 2 - 1)
            def _():
                pltpu.async_copy(table_hbm.at[sidx.at[slot, i + 2]],
                                 buf0, semA)

            hb.wait()
            pltpu.sync_copy(buf1, accum.at[srow.at[slot, i + 1]], add=True)
            return carry2
        lax.fori_loop(0, STG // 2, _step, 0)

        @pl.when(st < NSTG - 1)
        def _():
            pltpu.make_async_copy(gidx_hbm.at[pl.ds(nbase, STG)],
                                  sidx.at[nslot], semI).wait()
            pltpu.make_async_copy(grow_hbm.at[pl.ds(nbase, STG)],
                                  srow.at[nslot], semJ).wait()
        return carry
    lax.fori_loop(0, NSTG, _stage, 0)

    plsc.subcore_barrier()
    pltpu.sync_copy(accum.at[pl.ds(sbase, TPT)],
                    out_hbm.at[c, pl.ds(sbase, TPT)])


_sc_edge_cache = []


def _edge_accumulate(gidx, rowp, table):
    if not _sc_edge_cache:
        mesh = plsc.VectorSubcoreMesh(core_axis_name="c", subcore_axis_name="s")
        _sc_edge_cache.append(pl.kernel(
            _sc_edge_body,
            out_type=jax.ShapeDtypeStruct((2, ACC, C), jnp.float32),
            mesh=mesh,
            scratch_types=[
                pltpu.VMEM((2, STG, CH), jnp.int32),
                pltpu.VMEM((2, STG, CH), jnp.int32),
                pltpu.VMEM((CH, C), jnp.float32),
                pltpu.VMEM((CH, C), jnp.float32),
                pltpu.VMEM_SHARED((ACC, C), jnp.float32),
                pltpu.SemaphoreType.DMA,
                pltpu.SemaphoreType.DMA,
                pltpu.SemaphoreType.DMA,
                pltpu.SemaphoreType.DMA,
            ],
        ))
    zrow = jnp.zeros((CH, C), jnp.float32)
    return _sc_edge_cache[0](gidx, rowp, table, zrow)


def kernel(x, emb, edge_index, edge_type, batch_id,
           gamma1, beta1, W1, We, be, gamma2, beta2, W2):
    f32 = jnp.float32
    row = edge_index[0]
    colv = edge_index[1]
    pad = EPAD - E
    colp = jnp.concatenate([colv, jnp.zeros((pad,), jnp.int32)]).reshape(ROWS_I, CH)
    typep = jnp.concatenate([edge_type, jnp.zeros((pad,), jnp.int32)]).reshape(ROWS_I, CH)
    rowp = jnp.concatenate([row, jnp.full((pad,), TRASH, jnp.int32)]).reshape(ROWS_I, CH)
    bid2 = batch_id.reshape(N, 1)
    embp = jnp.concatenate([emb, jnp.zeros((8 - B, EMB), f32)], axis=0)
    g1r, b1r = gamma1.reshape(1, C), beta1.reshape(1, C)
    g2r, b2r = gamma2.reshape(1, C), beta2.reshape(1, C)
    ber = be.reshape(1, C)
    W1cat = W1.reshape(NET, C, C).transpose(1, 0, 2).reshape(C, NET * C)
    W2cat = W2.reshape(NET, C, C).transpose(1, 0, 2).reshape(C, NET * C)

    W1b = W1cat.astype(jnp.bfloat16)
    W2b = W2cat.astype(jnp.bfloat16)

    h1, embout, gidx = pl.pallas_call(
        _prep_body,
        out_shape=[
            jax.ShapeDtypeStruct((N, C), jnp.bfloat16),
            jax.ShapeDtypeStruct((8, C), f32),
            jax.ShapeDtypeStruct((ROWS_I, CH), jnp.int32),
        ],
    )(x, bid2, g1r, b1r, embp, We, ber, colp, typep)

    T1 = _matmul_call(h1, W1b)
    S1 = _edge_accumulate(gidx, rowp, T1.reshape(N * NET, C))

    h2 = pl.pallas_call(
        _mid_body,
        out_shape=jax.ShapeDtypeStruct((N, C), jnp.bfloat16),
    )(S1[0, :N], S1[1, :N], embout, bid2, g2r, b2r)

    T2 = _matmul_call(h2, W2b)
    S2 = _edge_accumulate(gidx, rowp, T2.reshape(N * NET, C))

    return pl.pallas_call(
        _final_body,
        out_shape=jax.ShapeDtypeStruct((N, C), f32),
    )(x, S2[0, :N], S2[1, :N])
